# Initial kernel scaffold; baseline (speedup 1.0000x reference)
import jax
import jax.numpy as jnp
from jax.experimental import pallas as pl


def kernel(sequence_repr_with_baseline, spike_indicators, Wr, br, W1r, b1r, W2r, b2r, W1s, b1s, W2s, b2s):
    raise NotImplementedError("write your pallas kernel here")


# speedup vs baseline: 1.5318x; 1.5318x over previous
import functools

import jax
import jax.numpy as jnp
from jax.experimental import pallas as pl

N_TOK = 4096
D_IN = 768
NUM_EXPERTS = 8
NUM_SPIKE = 2
TOTAL = NUM_EXPERTS + NUM_SPIKE
EXPERT_DIM = 768
SPIKE_CAP = 1536
OUT_DIM = 768
SPIKE_LEN = 16
BT = 256
NI = N_TOK // BT


def _router_body(x_ref, wr_ref, br_ref, spike_ref, wmat_ref):
    x = x_ref[...]
    logits = jnp.dot(x, wr_ref[...], preferred_element_type=jnp.float32)
    logits = logits + br_ref[0:1, :]
    lane = jax.lax.broadcasted_iota(jnp.int32, (BT, 128), 1)
    valid = lane < TOTAL
    spike_mask = jnp.logical_and(lane >= NUM_EXPERTS, valid).astype(jnp.float32)
    avg = jnp.sum(spike_ref[...], axis=1, keepdims=True) * (1.0 / SPIKE_LEN)
    adj = logits + avg * spike_mask
    adj = jnp.where(valid, adj, -1e30)
    m = jnp.max(adj, axis=1, keepdims=True)
    p = jnp.exp(adj - m) * valid.astype(jnp.float32)
    probs = p / jnp.sum(p, axis=1, keepdims=True)
    m1 = jnp.max(probs, axis=1, keepdims=True)
    i1 = jnp.min(jnp.where(probs >= m1, lane, 999), axis=1, keepdims=True)
    sel1 = lane == i1
    probs2 = jnp.where(sel1, -1.0, probs)
    m2 = jnp.max(probs2, axis=1, keepdims=True)
    i2 = jnp.min(jnp.where(probs2 >= m2, lane, 999), axis=1, keepdims=True)
    sel2 = lane == i2
    denom = m1 + m2 + 1e-9
    w = (m1 / denom) * sel1.astype(jnp.float32) + (m2 / denom) * sel2.astype(jnp.float32)
    wmat_ref[...] = w


def _expert_body(x_ref, w1_ref, w2_ref, b1_ref, b2_ref, wmat_ref, *rest,
                 e_base, has_partial):
    if has_partial:
        part_ref, out_ref = rest
    else:
        part_ref, (out_ref,) = None, rest
    e = pl.program_id(0)
    i = pl.program_id(1)

    @pl.when(jnp.logical_and(e == 0, i == 0))
    def _init():
        if has_partial:
            out_ref[...] = part_ref[...]
        else:
            out_ref[...] = jnp.zeros_like(out_ref)

    xb = x_ref[pl.ds(i * BT, BT), :]
    h = jnp.dot(xb, w1_ref[0], preferred_element_type=jnp.float32)
    h = jnp.maximum(h + b1_ref[0], 0.0).astype(jnp.bfloat16)
    y = jnp.dot(h, w2_ref[0], preferred_element_type=jnp.float32)
    y = y + b2_ref[0]
    lane = jax.lax.broadcasted_iota(jnp.int32, (BT, 128), 1)
    wblk = wmat_ref[pl.ds(i * BT, BT), :]
    wcol = jnp.sum(wblk * (lane == (e + e_base)).astype(jnp.float32), axis=1,
                   keepdims=True)
    out_ref[pl.ds(i * BT, BT), :] += wcol * y


def _expert_call(x_bf, w1, w2, b1, b2, wmat, partial, *, e_base):
    n_e = w1.shape[0]
    hid = w1.shape[2]
    has_partial = partial is not None
    whole_x = pl.BlockSpec((N_TOK, D_IN), lambda e, i: (0, 0))
    whole_o = pl.BlockSpec((N_TOK, OUT_DIM), lambda e, i: (0, 0))
    whole_w = pl.BlockSpec((N_TOK, 128), lambda e, i: (0, 0))
    in_specs = [
        whole_x,
        pl.BlockSpec((1, D_IN, hid), lambda e, i: (e, 0, 0)),
        pl.BlockSpec((1, hid, OUT_DIM), lambda e, i: (e, 0, 0)),
        pl.BlockSpec((1, 1, hid), lambda e, i: (e, 0, 0)),
        pl.BlockSpec((1, 1, OUT_DIM), lambda e, i: (e, 0, 0)),
        whole_w,
    ]
    args = [x_bf, w1, w2, b1[:, None, :], b2[:, None, :], wmat]
    if has_partial:
        in_specs.append(whole_o)
        args.append(partial)
    body = functools.partial(_expert_body, e_base=e_base, has_partial=has_partial)
    return pl.pallas_call(
        body,
        grid=(n_e, NI),
        in_specs=in_specs,
        out_specs=whole_o,
        out_shape=jax.ShapeDtypeStruct((N_TOK, OUT_DIM), jnp.float32),
    )(*args)


def kernel(sequence_repr_with_baseline, spike_indicators, Wr, br,
           W1r, b1r, W2r, b2r, W1s, b1s, W2s, b2s):
    x = sequence_repr_with_baseline
    wr_pad = jnp.zeros((D_IN, 128), jnp.float32).at[:, :TOTAL].set(Wr)
    br_pad = jnp.zeros((8, 128), jnp.float32).at[:, :TOTAL].set(
        jnp.broadcast_to(br, (8, TOTAL)))
    spike_pad = jnp.zeros((N_TOK, 128), jnp.float32).at[:, :SPIKE_LEN].set(
        spike_indicators)

    wmat = pl.pallas_call(
        _router_body,
        grid=(NI,),
        in_specs=[
            pl.BlockSpec((BT, D_IN), lambda i: (i, 0)),
            pl.BlockSpec((D_IN, 128), lambda i: (0, 0)),
            pl.BlockSpec((8, 128), lambda i: (0, 0)),
            pl.BlockSpec((BT, 128), lambda i: (i, 0)),
        ],
        out_specs=pl.BlockSpec((BT, 128), lambda i: (i, 0)),
        out_shape=jax.ShapeDtypeStruct((N_TOK, 128), jnp.float32),
    )(x, wr_pad, br_pad, spike_pad)

    x_bf = x.astype(jnp.bfloat16)
    out_r = _expert_call(x_bf, W1r.astype(jnp.bfloat16), W2r.astype(jnp.bfloat16),
                         b1r, b2r, wmat, None, e_base=0)
    out = _expert_call(x_bf, W1s.astype(jnp.bfloat16), W2s.astype(jnp.bfloat16),
                       b1s, b2s, wmat, out_r, e_base=NUM_EXPERTS)
    return out
